# baseline (speedup 1.0000x reference)
import dataclasses
import functools

import jax
import jax.numpy as jnp
from jax import lax
from jax.experimental import pallas as pl
from jax.experimental.pallas import tpu as pltpu
from jax.experimental.pallas import tpu_sc as plsc

N = 10000
E = 320000
D = 128
NC = 2
NS = 16
NW = NC * NS

CHUNK = 128

AGG_CHUNKS = 80
AGG_ROWS = 10240
AGG_ZROWS = AGG_ROWS // NS
AGG_OROWS = AGG_ROWS // NS

DEG_CHUNKS = 160
DEG_HROWS = 160
DEG_PROWS = 256

_vec_mesh = plsc.VectorSubcoreMesh(core_axis_name="c", subcore_axis_name="s")

_sc_params = pltpu.CompilerParams()
if "needs_layout_passes" in pltpu.CompilerParams.__dataclass_fields__:
    _sc_params = dataclasses.replace(_sc_params, needs_layout_passes=False)


@functools.partial(
    pl.kernel,
    out_type=jax.ShapeDtypeStruct((NC, DEG_PROWS, CHUNK), jnp.float32),
    mesh=_vec_mesh,
    scratch_types=[
        pltpu.VMEM((DEG_CHUNKS, CHUNK), jnp.int32),
        pltpu.VMEM((DEG_PROWS, CHUNK), jnp.float32),
        pltpu.VMEM((NS, 16, CHUNK), jnp.float32),
        pltpu.VMEM((16, CHUNK), jnp.float32),
        pltpu.VMEM_SHARED((NS, DEG_PROWS, CHUNK), jnp.float32),
    ],
    compiler_params=_sc_params,
)
def _deg_kernel(idx_hbm, out_hbm, idx_v, hist_v, comb_v, res_v, acc_sh):
    c = lax.axis_index("c")
    s = lax.axis_index("s")
    wid = s * NC + c

    zeros16 = jnp.zeros((16,), jnp.float32)
    ones16 = jnp.ones((16,), jnp.float32)

    @pl.loop(0, DEG_PROWS)
    def _(i):
        for k in range(CHUNK // 16):
            hist_v[i, pl.ds(k * 16, 16)] = zeros16

    pltpu.sync_copy(idx_hbm.at[wid], idx_v)

    @pl.loop(0, DEG_CHUNKS)
    def _(t):
        for k in range(CHUNK // 16):
            iv = idx_v[t, pl.ds(k * 16, 16)]
            hi = lax.shift_right_logical(iv, 7)
            lo = lax.bitwise_and(iv, 127)
            plsc.addupdate_scatter(hist_v, [hi, lo], ones16)

    pltpu.sync_copy(hist_v, acc_sh.at[s])
    plsc.subcore_barrier()
    for r in range(NS):
        pltpu.sync_copy(acc_sh.at[r, pl.ds(s * 16, 16)], comb_v.at[r])

    @pl.loop(0, 16)
    def _(t):
        for k in range(CHUNK // 16):
            acc = comb_v[0, t, pl.ds(k * 16, 16)]
            for r in range(1, NS):
                acc = acc + comb_v[r, t, pl.ds(k * 16, 16)]
            res_v[t, pl.ds(k * 16, 16)] = acc

    pltpu.sync_copy(res_v, out_hbm.at[c, pl.ds(s * 16, 16)])


def _agg_body(x_hbm, src_hbm, dst_hbm, out_hbm,
              src_v, dst_v, rows0, acc_sh):
    c = lax.axis_index("c")
    s = lax.axis_index("s")
    wid = s * NC + c

    @pl.loop(0, CHUNK)
    def _(i):
        for k in range(D // 16):
            rows0[i, pl.ds(k * 16, 16)] = jnp.zeros((16,), jnp.float32)

    zbase = s * AGG_ZROWS
    nfull = AGG_ZROWS // CHUNK
    rem = AGG_ZROWS - nfull * CHUNK
    for k in range(nfull):
        pltpu.sync_copy(rows0, acc_sh.at[pl.ds(zbase + k * CHUNK, CHUNK)])
    if rem:
        pltpu.sync_copy(rows0.at[pl.ds(0, rem)],
                        acc_sh.at[pl.ds(zbase + nfull * CHUNK, rem)])

    pltpu.sync_copy(src_hbm.at[wid], src_v)
    pltpu.sync_copy(dst_hbm.at[wid], dst_v)
    plsc.subcore_barrier()

    @pl.loop(0, AGG_CHUNKS)
    def _(j):
        pltpu.sync_copy(x_hbm.at[src_v.at[j]], rows0)
        pltpu.sync_copy(rows0, acc_sh.at[dst_v.at[j]], add=True)

    plsc.subcore_barrier()
    pltpu.sync_copy(acc_sh.at[pl.ds(s * AGG_OROWS, AGG_OROWS)],
                    out_hbm.at[c, pl.ds(s * AGG_OROWS, AGG_OROWS)])


_agg_kernel = functools.partial(
    pl.kernel,
    out_type=jax.ShapeDtypeStruct((NC, AGG_ROWS, D), jnp.float32),
    mesh=_vec_mesh,
    scratch_types=[
        pltpu.VMEM((AGG_CHUNKS, CHUNK), jnp.int32),
        pltpu.VMEM((AGG_CHUNKS, CHUNK), jnp.int32),
        pltpu.VMEM((CHUNK, D), jnp.float32),
        pltpu.VMEM_SHARED((AGG_ROWS, D), jnp.float32),
    ],
)(_agg_body)


R = 1000


def _prep_body(x_ref, dego_ref, degi_ref, xs_ref, nin_ref, nout_ref):
    do = dego_ref[0] + dego_ref[1]
    di = degi_ref[0] + degi_ref[1]
    no = jnp.where(do > 0, lax.rsqrt(jnp.maximum(do, 1e-12)), 0.0)
    ni = jnp.where(di > 0, lax.rsqrt(jnp.maximum(di, 1e-12)), 0.0)
    xs_ref[...] = x_ref[...] * no
    nout_ref[...] = no
    nin_ref[...] = ni


_prep = pl.pallas_call(
    _prep_body,
    grid=(N // R,),
    in_specs=[
        pl.BlockSpec((R, D), lambda i: (i, 0)),
        pl.BlockSpec((NC, R, 1), lambda i: (0, i, 0)),
        pl.BlockSpec((NC, R, 1), lambda i: (0, i + N // R, 0)),
    ],
    out_specs=[
        pl.BlockSpec((R, D), lambda i: (i, 0)),
        pl.BlockSpec((R, 1), lambda i: (i, 0)),
        pl.BlockSpec((R, 1), lambda i: (i, 0)),
    ],
    out_shape=[
        jax.ShapeDtypeStruct((N, D), jnp.float32),
        jax.ShapeDtypeStruct((N, 1), jnp.float32),
        jax.ShapeDtypeStruct((N, 1), jnp.float32),
    ],
)


def _dense1_body(agg_ref, nin_ref, nout_ref, w_ref, b_ref, o_ref):
    a = (agg_ref[0] + agg_ref[1]) * nin_ref[...]
    y = jnp.dot(a, w_ref[...], preferred_element_type=jnp.float32) + b_ref[...]
    o_ref[...] = jnp.maximum(y, 0.0) * nout_ref[...]


_dense1 = pl.pallas_call(
    _dense1_body,
    grid=(N // R,),
    in_specs=[
        pl.BlockSpec((NC, R, D), lambda i: (0, i, 0)),
        pl.BlockSpec((R, 1), lambda i: (i, 0)),
        pl.BlockSpec((R, 1), lambda i: (i, 0)),
        pl.BlockSpec((D, D), lambda i: (0, 0)),
        pl.BlockSpec((1, D), lambda i: (0, 0)),
    ],
    out_specs=pl.BlockSpec((R, D), lambda i: (i, 0)),
    out_shape=jax.ShapeDtypeStruct((N, D), jnp.float32),
)


def _dense2_body(agg_ref, nin_ref, w_ref, b_ref, o_ref):
    a = (agg_ref[0] + agg_ref[1]) * nin_ref[...]
    o_ref[...] = (jnp.dot(a, w_ref[...], preferred_element_type=jnp.float32)
                  + b_ref[...])


_dense2 = pl.pallas_call(
    _dense2_body,
    grid=(N // R,),
    in_specs=[
        pl.BlockSpec((NC, R, D), lambda i: (0, i, 0)),
        pl.BlockSpec((R, 1), lambda i: (i, 0)),
        pl.BlockSpec((D, D), lambda i: (0, 0)),
        pl.BlockSpec((1, D), lambda i: (0, 0)),
    ],
    out_specs=pl.BlockSpec((R, D), lambda i: (i, 0)),
    out_shape=jax.ShapeDtypeStruct((N, D), jnp.float32),
)


def kernel(features, edge_index, W1, b1, W2, b2):
    src = edge_index[0]
    dst = edge_index[1]

    deg_total = NW * DEG_CHUNKS * CHUNK
    deg_pad = deg_total - 2 * E
    deg_idx = jnp.concatenate([
        src, dst + N,
        2 * N + jnp.arange(deg_pad, dtype=jnp.int32) % (DEG_PROWS * CHUNK - 2 * N),
    ]).reshape(NW, DEG_CHUNKS, CHUNK)

    agg_total = NW * AGG_CHUNKS * CHUNK
    src_blk = jnp.concatenate([
        src, jnp.zeros((agg_total - E,), jnp.int32),
    ]).reshape(NW, AGG_CHUNKS, CHUNK)
    dst_blk = jnp.concatenate([
        dst, N + jnp.arange(agg_total - E, dtype=jnp.int32) % (AGG_ROWS - N),
    ]).reshape(NW, AGG_CHUNKS, CHUNK)

    b1r = b1.reshape(1, D)
    b2r = b2.reshape(1, D)

    degpart = _deg_kernel(deg_idx)
    degflat = degpart.reshape(NC, DEG_PROWS * CHUNK)[:, :2 * N]
    degflat = degflat.reshape(NC, 2 * N, 1)
    xs, nin, nout = _prep(features, degflat, degflat)
    agg1 = _agg_kernel(xs, src_blk, dst_blk)
    h1 = _dense1(agg1, nin, nout, W1, b1r)
    agg2 = _agg_kernel(h1, src_blk, dst_blk)
    return _dense2(agg2, nin, W2, b2r)


# speedup vs baseline: 1.0155x; 1.0155x over previous
import dataclasses
import functools

import jax
import jax.numpy as jnp
from jax import lax
from jax.experimental import pallas as pl
from jax.experimental.pallas import tpu as pltpu
from jax.experimental.pallas import tpu_sc as plsc

N = 10000
E = 320000
D = 128
NC = 2
NS = 16
NW = NC * NS

CHUNK = 128

AGG_CHUNKS = 80
AGG_ROWS = 10240
AGG_ZROWS = AGG_ROWS // NS
AGG_OROWS = AGG_ROWS // NS

DEG_CHUNKS = 160
DEG_HROWS = 160
DEG_PROWS = 256

_vec_mesh = plsc.VectorSubcoreMesh(core_axis_name="c", subcore_axis_name="s")

_sc_params = pltpu.CompilerParams()
if "needs_layout_passes" in pltpu.CompilerParams.__dataclass_fields__:
    _sc_params = dataclasses.replace(_sc_params, needs_layout_passes=False)


@functools.partial(
    pl.kernel,
    out_type=jax.ShapeDtypeStruct((NC, DEG_PROWS, CHUNK), jnp.float32),
    mesh=_vec_mesh,
    scratch_types=[
        pltpu.VMEM((DEG_CHUNKS, CHUNK), jnp.int32),
        pltpu.VMEM((DEG_PROWS, CHUNK), jnp.float32),
        pltpu.VMEM((NS, 16, CHUNK), jnp.float32),
        pltpu.VMEM((16, CHUNK), jnp.float32),
        pltpu.VMEM_SHARED((NS, DEG_PROWS, CHUNK), jnp.float32),
    ],
    compiler_params=_sc_params,
)
def _deg_kernel(idx_hbm, out_hbm, idx_v, hist_v, comb_v, res_v, acc_sh):
    c = lax.axis_index("c")
    s = lax.axis_index("s")
    wid = s * NC + c

    zeros16 = jnp.zeros((16,), jnp.float32)
    ones16 = jnp.ones((16,), jnp.float32)

    @pl.loop(0, DEG_PROWS)
    def _(i):
        for k in range(CHUNK // 16):
            hist_v[i, pl.ds(k * 16, 16)] = zeros16

    pltpu.sync_copy(idx_hbm.at[wid], idx_v)

    @pl.loop(0, DEG_CHUNKS)
    def _(t):
        for k in range(CHUNK // 16):
            iv = idx_v[t, pl.ds(k * 16, 16)]
            hi = lax.shift_right_logical(iv, 7)
            lo = lax.bitwise_and(iv, 127)
            plsc.addupdate_scatter(hist_v, [hi, lo], ones16)

    pltpu.sync_copy(hist_v, acc_sh.at[s])
    plsc.subcore_barrier()
    for r in range(NS):
        pltpu.sync_copy(acc_sh.at[r, pl.ds(s * 16, 16)], comb_v.at[r])

    @pl.loop(0, 16)
    def _(t):
        for k in range(CHUNK // 16):
            acc = comb_v[0, t, pl.ds(k * 16, 16)]
            for r in range(1, NS):
                acc = acc + comb_v[r, t, pl.ds(k * 16, 16)]
            res_v[t, pl.ds(k * 16, 16)] = acc

    pltpu.sync_copy(res_v, out_hbm.at[c, pl.ds(s * 16, 16)])


SUPER = 16
NBUF = 2


def _agg_body(x_hbm, src_hbm, dst_hbm, out_hbm,
              src_v, dst_v, buf0, buf1, gsem, acc_sh):
    bufs = (buf0, buf1)
    c = lax.axis_index("c")
    s = lax.axis_index("s")
    wid = s * NC + c

    @pl.loop(0, CHUNK)
    def _(i):
        for k in range(D // 16):
            buf0[i, pl.ds(k * 16, 16)] = jnp.zeros((16,), jnp.float32)

    zbase = s * AGG_ZROWS
    nfull = AGG_ZROWS // CHUNK
    for k in range(nfull):
        pltpu.sync_copy(buf0, acc_sh.at[pl.ds(zbase + k * CHUNK, CHUNK)])

    plsc.subcore_barrier()

    @pl.loop(0, AGG_CHUNKS, step=SUPER)
    def _(g):
        pltpu.sync_copy(src_hbm.at[wid].at[pl.ds(g, SUPER)], src_v)
        pltpu.sync_copy(dst_hbm.at[wid].at[pl.ds(g, SUPER)], dst_v)

        @pl.loop(0, SUPER, step=NBUF)
        def _(t):
            for b in range(NBUF):
                pltpu.async_copy(x_hbm.at[src_v.at[t + b]], bufs[b], gsem)
            for b in range(NBUF):
                pltpu.make_async_copy(x_hbm.at[src_v.at[t + b]], bufs[b],
                                      gsem).wait()
            for b in range(NBUF):
                pltpu.sync_copy(bufs[b], acc_sh.at[dst_v.at[t + b]], add=True)

    plsc.subcore_barrier()
    pltpu.sync_copy(acc_sh.at[pl.ds(s * AGG_OROWS, AGG_OROWS)],
                    out_hbm.at[c, pl.ds(s * AGG_OROWS, AGG_OROWS)])


_agg_kernel = functools.partial(
    pl.kernel,
    out_type=jax.ShapeDtypeStruct((NC, AGG_ROWS, D), jnp.float32),
    mesh=_vec_mesh,
    scratch_types=[
        pltpu.VMEM((SUPER, CHUNK), jnp.int32),
        pltpu.VMEM((SUPER, CHUNK), jnp.int32),
        pltpu.VMEM((CHUNK, D), jnp.float32),
        pltpu.VMEM((CHUNK, D), jnp.float32),
        pltpu.SemaphoreType.DMA,
        pltpu.VMEM_SHARED((AGG_ROWS, D), jnp.float32),
    ],
)(_agg_body)


R = 1000


def _prep_body(x_ref, dego_ref, degi_ref, xs_ref, nin_ref, nout_ref):
    do = dego_ref[0] + dego_ref[1]
    di = degi_ref[0] + degi_ref[1]
    no = jnp.where(do > 0, lax.rsqrt(jnp.maximum(do, 1e-12)), 0.0)
    ni = jnp.where(di > 0, lax.rsqrt(jnp.maximum(di, 1e-12)), 0.0)
    xs_ref[...] = x_ref[...] * no
    nout_ref[...] = no
    nin_ref[...] = ni


_prep = pl.pallas_call(
    _prep_body,
    grid=(N // R,),
    in_specs=[
        pl.BlockSpec((R, D), lambda i: (i, 0)),
        pl.BlockSpec((NC, R, 1), lambda i: (0, i, 0)),
        pl.BlockSpec((NC, R, 1), lambda i: (0, i + N // R, 0)),
    ],
    out_specs=[
        pl.BlockSpec((R, D), lambda i: (i, 0)),
        pl.BlockSpec((R, 1), lambda i: (i, 0)),
        pl.BlockSpec((R, 1), lambda i: (i, 0)),
    ],
    out_shape=[
        jax.ShapeDtypeStruct((N, D), jnp.float32),
        jax.ShapeDtypeStruct((N, 1), jnp.float32),
        jax.ShapeDtypeStruct((N, 1), jnp.float32),
    ],
)


def _dense1_body(agg_ref, nin_ref, nout_ref, w_ref, b_ref, o_ref):
    a = (agg_ref[0] + agg_ref[1]) * nin_ref[...]
    y = jnp.dot(a, w_ref[...], preferred_element_type=jnp.float32) + b_ref[...]
    o_ref[...] = jnp.maximum(y, 0.0) * nout_ref[...]


_dense1 = pl.pallas_call(
    _dense1_body,
    grid=(N // R,),
    in_specs=[
        pl.BlockSpec((NC, R, D), lambda i: (0, i, 0)),
        pl.BlockSpec((R, 1), lambda i: (i, 0)),
        pl.BlockSpec((R, 1), lambda i: (i, 0)),
        pl.BlockSpec((D, D), lambda i: (0, 0)),
        pl.BlockSpec((1, D), lambda i: (0, 0)),
    ],
    out_specs=pl.BlockSpec((R, D), lambda i: (i, 0)),
    out_shape=jax.ShapeDtypeStruct((N, D), jnp.float32),
)


def _dense2_body(agg_ref, nin_ref, w_ref, b_ref, o_ref):
    a = (agg_ref[0] + agg_ref[1]) * nin_ref[...]
    o_ref[...] = (jnp.dot(a, w_ref[...], preferred_element_type=jnp.float32)
                  + b_ref[...])


_dense2 = pl.pallas_call(
    _dense2_body,
    grid=(N // R,),
    in_specs=[
        pl.BlockSpec((NC, R, D), lambda i: (0, i, 0)),
        pl.BlockSpec((R, 1), lambda i: (i, 0)),
        pl.BlockSpec((D, D), lambda i: (0, 0)),
        pl.BlockSpec((1, D), lambda i: (0, 0)),
    ],
    out_specs=pl.BlockSpec((R, D), lambda i: (i, 0)),
    out_shape=jax.ShapeDtypeStruct((N, D), jnp.float32),
)


def kernel(features, edge_index, W1, b1, W2, b2):
    src = edge_index[0]
    dst = edge_index[1]

    deg_total = NW * DEG_CHUNKS * CHUNK
    deg_pad = deg_total - 2 * E
    deg_idx = jnp.concatenate([
        src, dst + N,
        2 * N + jnp.arange(deg_pad, dtype=jnp.int32) % (DEG_PROWS * CHUNK - 2 * N),
    ]).reshape(NW, DEG_CHUNKS, CHUNK)

    agg_total = NW * AGG_CHUNKS * CHUNK
    src_blk = jnp.concatenate([
        src, jnp.zeros((agg_total - E,), jnp.int32),
    ]).reshape(NW, AGG_CHUNKS, CHUNK)
    dst_blk = jnp.concatenate([
        dst, N + jnp.arange(agg_total - E, dtype=jnp.int32) % (AGG_ROWS - N),
    ]).reshape(NW, AGG_CHUNKS, CHUNK)

    b1r = b1.reshape(1, D)
    b2r = b2.reshape(1, D)

    degpart = _deg_kernel(deg_idx)
    degflat = degpart.reshape(NC, DEG_PROWS * CHUNK)[:, :2 * N]
    degflat = degflat.reshape(NC, 2 * N, 1)
    xs, nin, nout = _prep(features, degflat, degflat)
    agg1 = _agg_kernel(xs, src_blk, dst_blk)
    h1 = _dense1(agg1, nin, nout, W1, b1r)
    agg2 = _agg_kernel(h1, src_blk, dst_blk)
    return _dense2(agg2, nin, W2, b2r)
